# Initial kernel scaffold; baseline (speedup 1.0000x reference)
import jax
import jax.numpy as jnp
from jax.experimental import pallas as pl


def kernel(x, edge_index, batch, W1, b1, W2, b2, Wfc, bfc):
    raise NotImplementedError("write your pallas kernel here")


# speedup vs baseline: 4.6030x; 4.6030x over previous
import functools

import jax
import jax.numpy as jnp
from jax import lax
from jax.experimental import pallas as pl
from jax.experimental.pallas import tpu as pltpu
from jax.experimental.pallas import tpu_sc as plsc

NC, NS = 2, 16
NW = NC * NS
EK = 80
NUM_GRAPHS = 128


def _sc_aggregate(h, src, dst, zeros):
    N, F = h.shape
    E = src.shape[0]
    e_per_tile = E // NW
    n_chunks = e_per_tile // EK
    rpt = ((N + NS - 1) // NS + 7) // 8 * 8
    last = N - (NS - 1) * rpt
    mesh = plsc.VectorSubcoreMesh(core_axis_name="c", subcore_axis_name="s")

    @functools.partial(
        pl.kernel,
        out_type=jax.ShapeDtypeStruct((NC, N, F), jnp.float32),
        mesh=mesh,
        scratch_types=[
            pltpu.VMEM((EK,), jnp.int32),
            pltpu.VMEM((EK,), jnp.int32),
            pltpu.VMEM((EK, F), jnp.float32),
            pltpu.VMEM_SHARED((N, F), jnp.float32),
            pltpu.SemaphoreType.DMA,
        ],
    )
    def agg(h_hbm, src_hbm, dst_hbm, z_hbm, out_hbm, sidx, didx, rows, accum, sem):
        c = lax.axis_index("c")
        s = lax.axis_index("s")
        tid = c * NS + s
        @pl.when(s < NS - 1)
        def _():
            pltpu.sync_copy(z_hbm, accum.at[pl.ds(s * rpt, rpt)])

        @pl.when(s == NS - 1)
        def _():
            pltpu.sync_copy(z_hbm.at[pl.ds(0, last)],
                            accum.at[pl.ds(s * rpt, last)])

        plsc.subcore_barrier()

        base = tid * e_per_tile

        @pl.loop(0, n_chunks)
        def _chunk(k):
            eb = base + k * EK
            pltpu.sync_copy(src_hbm.at[pl.ds(eb, EK)], sidx)
            pltpu.sync_copy(dst_hbm.at[pl.ds(eb, EK)], didx)
            pltpu.async_copy(h_hbm.at[sidx], rows, sem).wait()
            pltpu.sync_copy(rows, accum.at[didx], add=True)

        plsc.subcore_barrier()

        @pl.when(s < NS - 1)
        def _():
            pltpu.sync_copy(accum.at[pl.ds(s * rpt, rpt)],
                            out_hbm.at[c, pl.ds(s * rpt, rpt)])

        @pl.when(s == NS - 1)
        def _():
            pltpu.sync_copy(accum.at[pl.ds(s * rpt, last)],
                            out_hbm.at[c, pl.ds(s * rpt, last)])

    return agg(h, src, dst, zeros)


def _mlp_layer(parts, h, w1, b1, w2, b2, blk=1000):
    N, F = h.shape

    def body(p_ref, h_ref, w1_ref, b1_ref, w2_ref, b2_ref, o_ref):
        z = p_ref[0] + p_ref[1] + h_ref[...]
        t = jnp.dot(z, w1_ref[...], preferred_element_type=jnp.float32)
        t = jnp.maximum(t + b1_ref[...], 0.0)
        o = jnp.dot(t, w2_ref[...], preferred_element_type=jnp.float32)
        o_ref[...] = jnp.maximum(o + b2_ref[...], 0.0)

    return pl.pallas_call(
        body,
        grid=(N // blk,),
        in_specs=[
            pl.BlockSpec((NC, blk, F), lambda i: (0, i, 0)),
            pl.BlockSpec((blk, F), lambda i: (i, 0)),
            pl.BlockSpec((F, F), lambda i: (0, 0)),
            pl.BlockSpec((1, F), lambda i: (0, 0)),
            pl.BlockSpec((F, F), lambda i: (0, 0)),
            pl.BlockSpec((1, F), lambda i: (0, 0)),
        ],
        out_specs=pl.BlockSpec((blk, F), lambda i: (i, 0)),
        out_shape=jax.ShapeDtypeStruct((N, F), jnp.float32),
    )(parts, h, w1, b1, w2, b2)


def _pool_fc(hs, batch3d, wfc, bfc, blk=1000):
    N, F = hs[0].shape
    nblk = N // blk
    Lh = len(hs)
    G = NUM_GRAPHS
    C = wfc.shape[-1]

    def body(b_ref, *refs):
        h_refs = refs[:Lh]
        wfc_ref, bfc_ref, o_ref, acc, cnt = refs[Lh:]
        i = pl.program_id(0)

        @pl.when(i == 0)
        def _():
            acc[...] = jnp.zeros_like(acc)
            cnt[...] = jnp.zeros_like(cnt)

        b = b_ref[0, 0, :]
        onehot = (b[:, None] == lax.broadcasted_iota(jnp.int32, (blk, G), 1))
        onehot = onehot.astype(jnp.float32)
        dn = (((0,), (0,)), ((), ()))
        for j in range(Lh):
            acc[j] += lax.dot_general(onehot, h_refs[j][...], dn,
                                      preferred_element_type=jnp.float32)
        cnt[...] += lax.dot_general(onehot, jnp.ones((blk, F), jnp.float32), dn,
                                    preferred_element_type=jnp.float32)

        @pl.when(i == nblk - 1)
        def _():
            recip = 1.0 / jnp.maximum(cnt[...], 1.0)
            logits = bfc_ref[...]
            for j in range(Lh):
                logits += jnp.dot(acc[j] * recip, wfc_ref[j],
                                  preferred_element_type=jnp.float32)
            m = jnp.max(logits, axis=-1, keepdims=True)
            ex = jnp.exp(logits - m)
            o_ref[...] = (logits - m) - jnp.log(jnp.sum(ex, axis=-1, keepdims=True))

    in_specs = [pl.BlockSpec((1, 1, blk), lambda i: (i, 0, 0))]
    in_specs += [pl.BlockSpec((blk, F), lambda i: (i, 0)) for _ in range(Lh)]
    in_specs += [
        pl.BlockSpec((Lh, F, C), lambda i: (0, 0, 0)),
        pl.BlockSpec((1, C), lambda i: (0, 0)),
    ]
    return pl.pallas_call(
        body,
        grid=(nblk,),
        in_specs=in_specs,
        out_specs=pl.BlockSpec((G, C), lambda i: (0, 0)),
        out_shape=jax.ShapeDtypeStruct((G, C), jnp.float32),
        scratch_shapes=[
            pltpu.VMEM((Lh, G, F), jnp.float32),
            pltpu.VMEM((G, F), jnp.float32),
        ],
    )(batch3d, *hs, wfc, bfc)


def kernel(x, edge_index, batch, W1, b1, W2, b2, Wfc, bfc):
    N, F = x.shape
    L = W1.shape[0]
    C = Wfc.shape[1]
    src = edge_index[0]
    dst = edge_index[1]
    zeros = jnp.zeros((((N + NS - 1) // NS + 7) // 8 * 8, F), jnp.float32)

    h = x
    hs = []
    for i in range(L):
        parts = _sc_aggregate(h, src, dst, zeros)
        h = _mlp_layer(parts, h, W1[i], b1[i].reshape(1, F),
                       W2[i], b2[i].reshape(1, F))
        hs.append(h)

    blk = 1000
    batch3d = batch.reshape(N // blk, 1, blk)
    wfc4 = Wfc.reshape(L, F, C)
    return _pool_fc(hs, batch3d, wfc4, bfc.reshape(1, C), blk=blk)


# speedup vs baseline: 10.6158x; 2.3063x over previous
import functools

import jax
import jax.numpy as jnp
from jax import lax
from jax.experimental import pallas as pl
from jax.experimental.pallas import tpu as pltpu
from jax.experimental.pallas import tpu_sc as plsc

NC, NS = 2, 16
NW = NC * NS
EK = 80
NUM_GRAPHS = 128


def _sc_aggregate(h, src2, dst3, zeros):
    N, F = h.shape
    e_per_tile = src2.shape[1]
    n_chunks = e_per_tile // EK
    assert n_chunks % 2 == 1
    rpt = ((N + NS - 1) // NS + 7) // 8 * 8
    last = N - (NS - 1) * rpt
    mesh = plsc.VectorSubcoreMesh(core_axis_name="c", subcore_axis_name="s")

    @functools.partial(
        pl.kernel,
        out_type=jax.ShapeDtypeStruct((NC, N, F), jnp.float32),
        mesh=mesh,
        scratch_types=[
            pltpu.VMEM((e_per_tile,), jnp.int32),
            pltpu.VMEM((n_chunks, EK), jnp.int32),
            pltpu.VMEM((EK, F), jnp.float32),
            pltpu.VMEM((EK, F), jnp.float32),
            pltpu.VMEM_SHARED((N, F), jnp.float32),
            pltpu.SemaphoreType.DMA,
            pltpu.SemaphoreType.DMA,
        ],
    )
    def agg(h_hbm, src_hbm, dst_hbm, z_hbm, out_hbm,
            sidx, didx, rows0, rows1, accum, sem0, sem1):
        c = lax.axis_index("c")
        s = lax.axis_index("s")
        tid = c * NS + s
        pltpu.sync_copy(src_hbm.at[tid], sidx)
        pltpu.sync_copy(dst_hbm.at[tid], didx)

        @pl.when(s < NS - 1)
        def _():
            pltpu.sync_copy(z_hbm, accum.at[pl.ds(s * rpt, rpt)])

        @pl.when(s == NS - 1)
        def _():
            pltpu.sync_copy(z_hbm.at[pl.ds(0, last)],
                            accum.at[pl.ds(s * rpt, last)])

        plsc.subcore_barrier()

        def gather(k, rows, sem):
            pltpu.async_copy(h_hbm.at[sidx.at[pl.ds(k * EK, EK)]], rows, sem)

        def wait(rows, sem):
            pltpu.make_async_copy(h_hbm.at[sidx.at[pl.ds(0, EK)]],
                                  rows, sem).wait()

        def scatter(k, rows):
            pltpu.sync_copy(rows, accum.at[didx.at[k]], add=True)

        gather(0, rows0, sem0)

        @pl.loop(0, (n_chunks - 1) // 2)
        def _pair(i):
            k = 2 * i
            gather(k + 1, rows1, sem1)
            wait(rows0, sem0)
            scatter(k, rows0)
            gather(k + 2, rows0, sem0)
            wait(rows1, sem1)
            scatter(k + 1, rows1)

        wait(rows0, sem0)
        scatter(n_chunks - 1, rows0)

        plsc.subcore_barrier()

        @pl.when(s < NS - 1)
        def _():
            pltpu.sync_copy(accum.at[pl.ds(s * rpt, rpt)],
                            out_hbm.at[c, pl.ds(s * rpt, rpt)])

        @pl.when(s == NS - 1)
        def _():
            pltpu.sync_copy(accum.at[pl.ds(s * rpt, last)],
                            out_hbm.at[c, pl.ds(s * rpt, last)])

    return agg(h, src2, dst3, zeros)


def _mlp_layer(parts, h, w1, b1, w2, b2, blk=1000):
    N, F = h.shape

    def body(p_ref, h_ref, w1_ref, b1_ref, w2_ref, b2_ref, o_ref):
        z = p_ref[0] + p_ref[1] + h_ref[...]
        t = jnp.dot(z, w1_ref[...], preferred_element_type=jnp.float32)
        t = jnp.maximum(t + b1_ref[...], 0.0)
        o = jnp.dot(t, w2_ref[...], preferred_element_type=jnp.float32)
        o_ref[...] = jnp.maximum(o + b2_ref[...], 0.0)

    return pl.pallas_call(
        body,
        grid=(N // blk,),
        in_specs=[
            pl.BlockSpec((NC, blk, F), lambda i: (0, i, 0)),
            pl.BlockSpec((blk, F), lambda i: (i, 0)),
            pl.BlockSpec((F, F), lambda i: (0, 0)),
            pl.BlockSpec((1, F), lambda i: (0, 0)),
            pl.BlockSpec((F, F), lambda i: (0, 0)),
            pl.BlockSpec((1, F), lambda i: (0, 0)),
        ],
        out_specs=pl.BlockSpec((blk, F), lambda i: (i, 0)),
        out_shape=jax.ShapeDtypeStruct((N, F), jnp.float32),
    )(parts, h, w1, b1, w2, b2)


def _pool_fc(hs, batch3d, wfc, bfc, blk=1000):
    N, F = hs[0].shape
    nblk = N // blk
    Lh = len(hs)
    G = NUM_GRAPHS
    C = wfc.shape[-1]

    def body(b_ref, *refs):
        h_refs = refs[:Lh]
        wfc_ref, bfc_ref, o_ref, acc, cnt = refs[Lh:]
        i = pl.program_id(0)

        @pl.when(i == 0)
        def _():
            acc[...] = jnp.zeros_like(acc)
            cnt[...] = jnp.zeros_like(cnt)

        b = b_ref[0, 0, :]
        onehot = (b[:, None] == lax.broadcasted_iota(jnp.int32, (blk, G), 1))
        onehot = onehot.astype(jnp.float32)
        dn = (((0,), (0,)), ((), ()))
        for j in range(Lh):
            acc[j] += lax.dot_general(onehot, h_refs[j][...], dn,
                                      preferred_element_type=jnp.float32)
        cnt[...] += lax.dot_general(onehot, jnp.ones((blk, F), jnp.float32), dn,
                                    preferred_element_type=jnp.float32)

        @pl.when(i == nblk - 1)
        def _():
            recip = 1.0 / jnp.maximum(cnt[...], 1.0)
            logits = bfc_ref[...]
            for j in range(Lh):
                logits += jnp.dot(acc[j] * recip, wfc_ref[j],
                                  preferred_element_type=jnp.float32)
            m = jnp.max(logits, axis=-1, keepdims=True)
            ex = jnp.exp(logits - m)
            o_ref[...] = (logits - m) - jnp.log(jnp.sum(ex, axis=-1, keepdims=True))

    in_specs = [pl.BlockSpec((1, 1, blk), lambda i: (i, 0, 0))]
    in_specs += [pl.BlockSpec((blk, F), lambda i: (i, 0)) for _ in range(Lh)]
    in_specs += [
        pl.BlockSpec((Lh, F, C), lambda i: (0, 0, 0)),
        pl.BlockSpec((1, C), lambda i: (0, 0)),
    ]
    return pl.pallas_call(
        body,
        grid=(nblk,),
        in_specs=in_specs,
        out_specs=pl.BlockSpec((G, C), lambda i: (0, 0)),
        out_shape=jax.ShapeDtypeStruct((G, C), jnp.float32),
        scratch_shapes=[
            pltpu.VMEM((Lh, G, F), jnp.float32),
            pltpu.VMEM((G, F), jnp.float32),
        ],
    )(batch3d, *hs, wfc, bfc)


def kernel(x, edge_index, batch, W1, b1, W2, b2, Wfc, bfc):
    N, F = x.shape
    L = W1.shape[0]
    C = Wfc.shape[1]
    E = edge_index.shape[1]
    e_per_tile = E // NW
    n_chunks = e_per_tile // EK
    src2 = edge_index[0].reshape(NW, e_per_tile)
    dst3 = edge_index[1].reshape(NW, n_chunks, EK)
    zeros = jnp.zeros((((N + NS - 1) // NS + 7) // 8 * 8, F), jnp.float32)

    h = x
    hs = []
    for i in range(L):
        parts = _sc_aggregate(h, src2, dst3, zeros)
        h = _mlp_layer(parts, h, W1[i], b1[i].reshape(1, F),
                       W2[i], b2[i].reshape(1, F))
        hs.append(h)

    blk = 1000
    batch3d = batch.reshape(N // blk, 1, blk)
    wfc4 = Wfc.reshape(L, F, C)
    return _pool_fc(hs, batch3d, wfc4, bfc.reshape(1, C), blk=blk)


# speedup vs baseline: 10.6772x; 1.0058x over previous
import functools

import jax
import jax.numpy as jnp
from jax import lax
from jax.experimental import pallas as pl
from jax.experimental.pallas import tpu as pltpu
from jax.experimental.pallas import tpu_sc as plsc

NC, NS = 2, 16
NW = NC * NS
EK = 80
NUM_GRAPHS = 128


def _sc_aggregate(h, src2, dst3, zeros):
    N, F = h.shape
    e_per_tile = src2.shape[1]
    n_chunks = e_per_tile // EK
    assert n_chunks % 2 == 1 and n_chunks >= 5
    rpt = ((N + NS - 1) // NS + 7) // 8 * 8
    last = N - (NS - 1) * rpt
    mesh = plsc.VectorSubcoreMesh(core_axis_name="c", subcore_axis_name="s")

    @functools.partial(
        pl.kernel,
        out_type=jax.ShapeDtypeStruct((NC, N, F), jnp.float32),
        mesh=mesh,
        scratch_types=[
            pltpu.VMEM((e_per_tile,), jnp.int32),
            pltpu.VMEM((n_chunks, EK), jnp.int32),
            pltpu.VMEM((EK, F), jnp.float32),
            pltpu.VMEM((EK, F), jnp.float32),
            pltpu.VMEM_SHARED((N, F), jnp.float32),
            pltpu.SemaphoreType.DMA,
            pltpu.SemaphoreType.DMA,
        ],
    )
    def agg(h_hbm, src_hbm, dst_hbm, z_hbm, out_hbm,
            sidx, didx, rows0, rows1, accum, sem0, sem1):
        c = lax.axis_index("c")
        s = lax.axis_index("s")
        tid = c * NS + s

        def gather(k, rows, sem):
            pltpu.async_copy(h_hbm.at[sidx.at[pl.ds(k * EK, EK)]], rows, sem)

        def wait(rows, sem):
            pltpu.make_async_copy(h_hbm.at[sidx.at[pl.ds(0, EK)]],
                                  rows, sem).wait()

        def scatter(k, rows):
            pltpu.sync_copy(rows, accum.at[didx.at[k]], add=True)

        pltpu.sync_copy(src_hbm.at[tid], sidx)
        pltpu.sync_copy(dst_hbm.at[tid], didx)
        gather(0, rows0, sem0)

        @pl.when(s < NS - 1)
        def _():
            pltpu.sync_copy(z_hbm, accum.at[pl.ds(s * rpt, rpt)])

        @pl.when(s == NS - 1)
        def _():
            pltpu.sync_copy(z_hbm.at[pl.ds(0, last)],
                            accum.at[pl.ds(s * rpt, last)])

        plsc.subcore_barrier()

        @pl.loop(0, (n_chunks - 1) // 2)
        def _pair(i):
            k = 2 * i
            gather(k + 1, rows1, sem1)
            wait(rows0, sem0)
            scatter(k, rows0)
            gather(k + 2, rows0, sem0)
            wait(rows1, sem1)
            scatter(k + 1, rows1)

        wait(rows0, sem0)
        scatter(n_chunks - 1, rows0)

        plsc.subcore_barrier()

        @pl.when(s < NS - 1)
        def _():
            pltpu.sync_copy(accum.at[pl.ds(s * rpt, rpt)],
                            out_hbm.at[c, pl.ds(s * rpt, rpt)])

        @pl.when(s == NS - 1)
        def _():
            pltpu.sync_copy(accum.at[pl.ds(s * rpt, last)],
                            out_hbm.at[c, pl.ds(s * rpt, last)])

    return agg(h, src2, dst3, zeros)


def _mlp_layer(parts, h, w1, b1, w2, b2, blk=1000):
    N, F = h.shape

    def body(p_ref, h_ref, w1_ref, b1_ref, w2_ref, b2_ref, o_ref):
        z = p_ref[0] + p_ref[1] + h_ref[...]
        t = jnp.dot(z, w1_ref[...], preferred_element_type=jnp.float32)
        t = jnp.maximum(t + b1_ref[...], 0.0)
        o = jnp.dot(t, w2_ref[...], preferred_element_type=jnp.float32)
        o_ref[...] = jnp.maximum(o + b2_ref[...], 0.0)

    return pl.pallas_call(
        body,
        grid=(N // blk,),
        in_specs=[
            pl.BlockSpec((NC, blk, F), lambda i: (0, i, 0)),
            pl.BlockSpec((blk, F), lambda i: (i, 0)),
            pl.BlockSpec((F, F), lambda i: (0, 0)),
            pl.BlockSpec((1, F), lambda i: (0, 0)),
            pl.BlockSpec((F, F), lambda i: (0, 0)),
            pl.BlockSpec((1, F), lambda i: (0, 0)),
        ],
        out_specs=pl.BlockSpec((blk, F), lambda i: (i, 0)),
        out_shape=jax.ShapeDtypeStruct((N, F), jnp.float32),
    )(parts, h, w1, b1, w2, b2)


def _pool_fc(hs, batch3d, wfc, bfc, blk=1000):
    N, F = hs[0].shape
    nblk = N // blk
    Lh = len(hs)
    G = NUM_GRAPHS
    C = wfc.shape[-1]

    def body(b_ref, *refs):
        h_refs = refs[:Lh]
        wfc_ref, bfc_ref, o_ref, acc, cnt = refs[Lh:]
        i = pl.program_id(0)

        @pl.when(i == 0)
        def _():
            acc[...] = jnp.zeros_like(acc)
            cnt[...] = jnp.zeros_like(cnt)

        b = b_ref[0, 0, :]
        onehot = (b[:, None] == lax.broadcasted_iota(jnp.int32, (blk, G), 1))
        onehot = onehot.astype(jnp.float32)
        dn = (((0,), (0,)), ((), ()))
        for j in range(Lh):
            acc[j] += lax.dot_general(onehot, h_refs[j][...], dn,
                                      preferred_element_type=jnp.float32)
        cnt[...] += lax.dot_general(onehot, jnp.ones((blk, F), jnp.float32), dn,
                                    preferred_element_type=jnp.float32)

        @pl.when(i == nblk - 1)
        def _():
            recip = 1.0 / jnp.maximum(cnt[...], 1.0)
            logits = bfc_ref[...]
            for j in range(Lh):
                logits += jnp.dot(acc[j] * recip, wfc_ref[j],
                                  preferred_element_type=jnp.float32)
            m = jnp.max(logits, axis=-1, keepdims=True)
            ex = jnp.exp(logits - m)
            o_ref[...] = (logits - m) - jnp.log(jnp.sum(ex, axis=-1, keepdims=True))

    in_specs = [pl.BlockSpec((1, 1, blk), lambda i: (i, 0, 0))]
    in_specs += [pl.BlockSpec((blk, F), lambda i: (i, 0)) for _ in range(Lh)]
    in_specs += [
        pl.BlockSpec((Lh, F, C), lambda i: (0, 0, 0)),
        pl.BlockSpec((1, C), lambda i: (0, 0)),
    ]
    return pl.pallas_call(
        body,
        grid=(nblk,),
        in_specs=in_specs,
        out_specs=pl.BlockSpec((G, C), lambda i: (0, 0)),
        out_shape=jax.ShapeDtypeStruct((G, C), jnp.float32),
        scratch_shapes=[
            pltpu.VMEM((Lh, G, F), jnp.float32),
            pltpu.VMEM((G, F), jnp.float32),
        ],
    )(batch3d, *hs, wfc, bfc)


def kernel(x, edge_index, batch, W1, b1, W2, b2, Wfc, bfc):
    N, F = x.shape
    L = W1.shape[0]
    C = Wfc.shape[1]
    E = edge_index.shape[1]
    e_per_tile = E // NW
    n_chunks = e_per_tile // EK
    src2 = edge_index[0].reshape(NW, e_per_tile)
    dst3 = edge_index[1].reshape(NW, n_chunks, EK)
    zeros = jnp.zeros((((N + NS - 1) // NS + 7) // 8 * 8, F), jnp.float32)

    h = x
    hs = []
    for i in range(L):
        parts = _sc_aggregate(h, src2, dst3, zeros)
        h = _mlp_layer(parts, h, W1[i], b1[i].reshape(1, F),
                       W2[i], b2[i].reshape(1, F))
        hs.append(h)

    blk = 1000
    batch3d = batch.reshape(N // blk, 1, blk)
    wfc4 = Wfc.reshape(L, F, C)
    return _pool_fc(hs, batch3d, wfc4, bfc.reshape(1, C), blk=blk)


# speedup vs baseline: 11.0915x; 1.0388x over previous
import functools

import jax
import jax.numpy as jnp
from jax import lax
from jax.experimental import pallas as pl
from jax.experimental.pallas import tpu as pltpu
from jax.experimental.pallas import tpu_sc as plsc

NC, NS = 2, 16
NW = NC * NS
EK = 80
NUM_GRAPHS = 128


def _sc_aggregate(h, src2, dst3, zeros):
    N, F = h.shape
    e_per_tile = src2.shape[1]
    n_chunks = e_per_tile // EK
    assert n_chunks % 2 == 1 and n_chunks >= 5
    rpt = ((N + NS - 1) // NS + 7) // 8 * 8
    last = N - (NS - 1) * rpt
    mesh = plsc.VectorSubcoreMesh(core_axis_name="c", subcore_axis_name="s")

    @functools.partial(
        pl.kernel,
        out_type=jax.ShapeDtypeStruct((NC, N, F), jnp.float32),
        mesh=mesh,
        scratch_types=[
            pltpu.VMEM((e_per_tile,), jnp.int32),
            pltpu.VMEM((n_chunks, EK), jnp.int32),
            pltpu.VMEM((EK, F), jnp.float32),
            pltpu.VMEM((EK, F), jnp.float32),
            pltpu.VMEM_SHARED((N, F), jnp.float32),
            pltpu.SemaphoreType.DMA,
            pltpu.SemaphoreType.DMA,
        ],
    )
    def agg(h_hbm, src_hbm, dst_hbm, z_hbm, out_hbm,
            sidx, didx, rows0, rows1, accum, sem0, sem1):
        c = lax.axis_index("c")
        s = lax.axis_index("s")
        tid = c * NS + s

        def gather(k, rows, sem):
            pltpu.async_copy(h_hbm.at[sidx.at[pl.ds(k * EK, EK)]], rows, sem)

        def wait(rows, sem):
            pltpu.make_async_copy(h_hbm.at[sidx.at[pl.ds(0, EK)]],
                                  rows, sem).wait()

        def scatter(k, rows):
            pltpu.sync_copy(rows, accum.at[didx.at[k]], add=True)

        pltpu.sync_copy(src_hbm.at[tid], sidx)
        pltpu.sync_copy(dst_hbm.at[tid], didx)
        gather(0, rows0, sem0)

        @pl.when(s < NS - 1)
        def _():
            pltpu.sync_copy(z_hbm, accum.at[pl.ds(s * rpt, rpt)])

        @pl.when(s == NS - 1)
        def _():
            pltpu.sync_copy(z_hbm.at[pl.ds(0, last)],
                            accum.at[pl.ds(s * rpt, last)])

        plsc.subcore_barrier()

        @pl.loop(0, (n_chunks - 1) // 2, unroll=2)
        def _pair(i):
            k = 2 * i
            gather(k + 1, rows1, sem1)
            wait(rows0, sem0)
            scatter(k, rows0)
            gather(k + 2, rows0, sem0)
            wait(rows1, sem1)
            scatter(k + 1, rows1)

        wait(rows0, sem0)
        scatter(n_chunks - 1, rows0)

        plsc.subcore_barrier()

        @pl.when(s < NS - 1)
        def _():
            pltpu.sync_copy(accum.at[pl.ds(s * rpt, rpt)],
                            out_hbm.at[c, pl.ds(s * rpt, rpt)])

        @pl.when(s == NS - 1)
        def _():
            pltpu.sync_copy(accum.at[pl.ds(s * rpt, last)],
                            out_hbm.at[c, pl.ds(s * rpt, last)])

    return agg(h, src2, dst3, zeros)


def _mlp_layer(parts, h, w1, b1, w2, b2, blk=2000):
    N, F = h.shape

    def body(p_ref, h_ref, w1_ref, b1_ref, w2_ref, b2_ref, o_ref):
        z = p_ref[0] + p_ref[1] + h_ref[...]
        t = jnp.dot(z, w1_ref[...], preferred_element_type=jnp.float32)
        t = jnp.maximum(t + b1_ref[...], 0.0)
        o = jnp.dot(t, w2_ref[...], preferred_element_type=jnp.float32)
        o_ref[...] = jnp.maximum(o + b2_ref[...], 0.0)

    return pl.pallas_call(
        body,
        grid=(N // blk,),
        in_specs=[
            pl.BlockSpec((NC, blk, F), lambda i: (0, i, 0)),
            pl.BlockSpec((blk, F), lambda i: (i, 0)),
            pl.BlockSpec((F, F), lambda i: (0, 0)),
            pl.BlockSpec((1, F), lambda i: (0, 0)),
            pl.BlockSpec((F, F), lambda i: (0, 0)),
            pl.BlockSpec((1, F), lambda i: (0, 0)),
        ],
        out_specs=pl.BlockSpec((blk, F), lambda i: (i, 0)),
        out_shape=jax.ShapeDtypeStruct((N, F), jnp.float32),
    )(parts, h, w1, b1, w2, b2)


def _mlp_pool_fc(parts, h, w1, b1, w2, b2, hs, batch3d, wfc, bfc, blk=2000):
    N, F = h.shape
    nblk = N // blk
    Lh = len(hs) + 1
    G = NUM_GRAPHS
    C = wfc.shape[-1]

    def body(b_ref, p_ref, h_ref, w1_ref, b1_ref, w2_ref, b2_ref,
             h1_ref, h2_ref, h3_ref, wfc_ref, bfc_ref, o_ref, acc, cnt):
        i = pl.program_id(0)

        @pl.when(i == 0)
        def _():
            acc[...] = jnp.zeros_like(acc)
            cnt[...] = jnp.zeros_like(cnt)

        z = p_ref[0] + p_ref[1] + h_ref[...]
        t = jnp.dot(z, w1_ref[...], preferred_element_type=jnp.float32)
        t = jnp.maximum(t + b1_ref[...], 0.0)
        h4 = jnp.dot(t, w2_ref[...], preferred_element_type=jnp.float32)
        h4 = jnp.maximum(h4 + b2_ref[...], 0.0)

        b = b_ref[0, 0, :]
        onehot = (b[:, None] == lax.broadcasted_iota(jnp.int32, (blk, G), 1))
        onehot = onehot.astype(jnp.float32)
        dn = (((0,), (0,)), ((), ()))
        for j, hj in enumerate((h1_ref[...], h2_ref[...], h3_ref[...], h4)):
            acc[j] += lax.dot_general(onehot, hj, dn,
                                      preferred_element_type=jnp.float32)
        cnt[...] += lax.dot_general(onehot, jnp.ones((blk, F), jnp.float32), dn,
                                    preferred_element_type=jnp.float32)

        @pl.when(i == nblk - 1)
        def _():
            recip = 1.0 / jnp.maximum(cnt[...], 1.0)
            logits = bfc_ref[...]
            for j in range(Lh):
                logits += jnp.dot(acc[j] * recip, wfc_ref[j],
                                  preferred_element_type=jnp.float32)
            m = jnp.max(logits, axis=-1, keepdims=True)
            ex = jnp.exp(logits - m)
            o_ref[...] = (logits - m) - jnp.log(jnp.sum(ex, axis=-1, keepdims=True))

    in_specs = [
        pl.BlockSpec((1, 1, blk), lambda i: (i, 0, 0)),
        pl.BlockSpec((NC, blk, F), lambda i: (0, i, 0)),
        pl.BlockSpec((blk, F), lambda i: (i, 0)),
        pl.BlockSpec((F, F), lambda i: (0, 0)),
        pl.BlockSpec((1, F), lambda i: (0, 0)),
        pl.BlockSpec((F, F), lambda i: (0, 0)),
        pl.BlockSpec((1, F), lambda i: (0, 0)),
    ]
    in_specs += [pl.BlockSpec((blk, F), lambda i: (i, 0)) for _ in hs]
    in_specs += [
        pl.BlockSpec((Lh, F, C), lambda i: (0, 0, 0)),
        pl.BlockSpec((1, C), lambda i: (0, 0)),
    ]
    return pl.pallas_call(
        body,
        grid=(nblk,),
        in_specs=in_specs,
        out_specs=pl.BlockSpec((G, C), lambda i: (0, 0)),
        out_shape=jax.ShapeDtypeStruct((G, C), jnp.float32),
        scratch_shapes=[
            pltpu.VMEM((Lh, G, F), jnp.float32),
            pltpu.VMEM((G, F), jnp.float32),
        ],
    )(batch3d, parts, h, w1, b1, w2, b2, *hs, wfc, bfc)


def kernel(x, edge_index, batch, W1, b1, W2, b2, Wfc, bfc):
    N, F = x.shape
    L = W1.shape[0]
    C = Wfc.shape[1]
    E = edge_index.shape[1]
    e_per_tile = E // NW
    n_chunks = e_per_tile // EK
    src2 = edge_index[0].reshape(NW, e_per_tile)
    dst3 = edge_index[1].reshape(NW, n_chunks, EK)
    zeros = jnp.zeros((((N + NS - 1) // NS + 7) // 8 * 8, F), jnp.float32)

    h = x
    hs = []
    for i in range(L - 1):
        parts = _sc_aggregate(h, src2, dst3, zeros)
        h = _mlp_layer(parts, h, W1[i], b1[i].reshape(1, F),
                       W2[i], b2[i].reshape(1, F))
        hs.append(h)

    parts = _sc_aggregate(h, src2, dst3, zeros)
    blk = 2000
    batch3d = batch.reshape(N // blk, 1, blk)
    wfc4 = Wfc.reshape(L, F, C)
    return _mlp_pool_fc(parts, h, W1[L - 1], b1[L - 1].reshape(1, F),
                        W2[L - 1], b2[L - 1].reshape(1, F),
                        hs, batch3d, wfc4, bfc.reshape(1, C), blk=blk)
